# baseline (speedup 1.0000x reference)
import jax
import jax.numpy as jnp
from jax import lax
from jax.experimental import pallas as pl
from jax.experimental.pallas import tpu as pltpu
from jax.experimental.pallas import tpu_sc as plsc

N = 10000
E = 320000
D = 128
H = 128
C = 47

NC = 2
NS = 16
NWORK = NC * NS
EPW = E // NWORK
CH = 80
NCHUNK = EPW // CH
PUB = 1000
NPUB = N // PUB
ZR = 40
NZ = PUB // ZR


CHB = 50
NCB = EPW // CHB
NSEG = 4
SEG = NCB // NSEG
NBUF = 4


def _sc_agg_body(y_hbm, src_hbm, dst_hbm, out_hbm, srcv, dstv, rows, semg,
                 sems, agg_sh):
    cid = lax.axis_index("c")
    sid = lax.axis_index("s")
    wid = cid * NS + sid

    def _zero_row(i, _):
        for j in range(H // 16):
            rows[0, i, pl.ds(j * 16, 16)] = jnp.zeros((16,), jnp.float32)
        return 0
    lax.fori_loop(0, ZR, _zero_row, 0)
    r0 = sid * PUB

    @pl.when(sid < NPUB)
    def _():
        for k in range(NZ):
            pltpu.sync_copy(rows.at[0, pl.ds(0, ZR), :],
                            agg_sh.at[pl.ds(r0 + k * ZR, ZR), :])
    plsc.subcore_barrier()

    def _startg(b, r):
        pltpu.async_copy(y_hbm.at[srcv.at[r]], rows.at[b], semg.at[b])

    def _waitg(b):
        pltpu.make_async_copy(y_hbm.at[srcv.at[0]], rows.at[b],
                              semg.at[b]).wait()

    def _scat(b, c):
        pltpu.async_copy(rows.at[b], agg_sh.at[dstv.at[c]], sems.at[b],
                         add=True)

    def _waits(b):
        pltpu.make_async_copy(rows.at[b], agg_sh.at[dstv.at[0]],
                              sems.at[b]).wait()

    def _seg(h):
        pltpu.sync_copy(src_hbm.at[wid * NSEG + h], srcv)
        pltpu.sync_copy(dst_hbm.at[wid * NSEG + h], dstv)

        def _body(c, _):
            @pl.when(c < SEG)
            def _():
                b = lax.rem(c, NBUF)

                @pl.when(c >= NBUF)
                def _():
                    _waits(b)
                _startg(b, c)

            @pl.when(c >= 2)
            def _():
                b2 = lax.rem(c - 2, NBUF)
                _waitg(b2)
                _scat(b2, c - 2)
            return 0
        lax.fori_loop(0, SEG + 2, _body, 0)
        for b in range(NBUF):
            _waits(b)

    for h in range(NSEG):
        _seg(h)
    plsc.subcore_barrier()

    @pl.when(sid < NPUB)
    def _():
        sl = pl.ds(r0, PUB)
        pltpu.sync_copy(agg_sh.at[sl, :], out_hbm.at[cid, sl, :])


_sc_agg = pl.kernel(
    _sc_agg_body,
    out_type=jax.ShapeDtypeStruct((NC, N, H), jnp.float32),
    mesh=plsc.VectorSubcoreMesh(core_axis_name="c", subcore_axis_name="s"),
    scratch_types=[
        pltpu.VMEM((SEG, CHB), jnp.int32),
        pltpu.VMEM((SEG, CHB), jnp.int32),
        pltpu.VMEM((NBUF, CHB, H), jnp.float32),
        pltpu.SemaphoreType.DMA((NBUF,)),
        pltpu.SemaphoreType.DMA((NBUF,)),
        pltpu.VMEM_SHARED((N, H), jnp.float32),
    ],
)


def _sc_deg_body(dst_hbm, deg_hbm, dstv, hist):
    cid = lax.axis_index("c")
    sid = lax.axis_index("s")
    wid = cid * NS + sid

    def _zero(i, _):
        hist[pl.ds(i * 16, 16)] = jnp.zeros((16,), jnp.float32)
        return 0
    lax.fori_loop(0, N // 16, _zero, 0)

    pltpu.sync_copy(dst_hbm.at[wid], dstv)
    ones = jnp.ones((16,), jnp.float32)

    def _edges(i, _):
        def _vec(k, _):
            idx = dstv[i, pl.ds(k * 16, 16)]
            plsc.addupdate_scatter(hist, [idx], ones)
            return 0
        lax.fori_loop(0, CH // 16, _vec, 0)
        return 0
    lax.fori_loop(0, NCHUNK, _edges, 0)

    pltpu.sync_copy(hist, deg_hbm.at[wid])


_sc_deg = pl.kernel(
    _sc_deg_body,
    out_type=jax.ShapeDtypeStruct((NWORK, N), jnp.float32),
    mesh=plsc.VectorSubcoreMesh(core_axis_name="c", subcore_axis_name="s"),
    scratch_types=[
        pltpu.VMEM((NCHUNK, CH), jnp.int32),
        pltpu.VMEM((N,), jnp.float32),
    ],
    compiler_params=pltpu.CompilerParams(needs_layout_passes=False),
)


_BLK = 1000
_GRID = N // _BLK


def _mm0_body(x_ref, wn_ref, ws_ref, y_ref, s_ref):
    x = x_ref[...]
    y_ref[...] = jnp.dot(x, wn_ref[...].T, preferred_element_type=jnp.float32)
    s_ref[...] = jnp.dot(x, ws_ref[...].T, preferred_element_type=jnp.float32)


def _mm0(x, wn, ws):
    return pl.pallas_call(
        _mm0_body,
        grid=(_GRID,),
        in_specs=[
            pl.BlockSpec((_BLK, D), lambda i: (i, 0)),
            pl.BlockSpec((H, D), lambda i: (0, 0)),
            pl.BlockSpec((H, D), lambda i: (0, 0)),
        ],
        out_specs=[
            pl.BlockSpec((_BLK, H), lambda i: (i, 0)),
            pl.BlockSpec((_BLK, H), lambda i: (i, 0)),
        ],
        out_shape=[jax.ShapeDtypeStruct((N, H), jnp.float32),
                   jax.ShapeDtypeStruct((N, H), jnp.float32)],
    )(x, wn, ws)


def _mid_body(s_ref, agg_ref, deg_ref, b_ref, wn_ref, ws_ref, y_ref, sn_ref):
    agg = agg_ref[0] + agg_ref[1]
    deg = jnp.sum(deg_ref[...], axis=1, keepdims=True)
    inv = 1.0 / jnp.maximum(deg, 1.0)
    h = jnp.maximum(s_ref[...] + agg * inv + b_ref[...], 0.0)
    y_ref[...] = jnp.dot(h, wn_ref[...].T, preferred_element_type=jnp.float32)
    sn_ref[...] = jnp.dot(h, ws_ref[...].T, preferred_element_type=jnp.float32)


def _mid(s, aggp, degp, b, wn, ws):
    wo = wn.shape[0]
    return pl.pallas_call(
        _mid_body,
        grid=(_GRID,),
        in_specs=[
            pl.BlockSpec((_BLK, H), lambda i: (i, 0)),
            pl.BlockSpec((NC, _BLK, H), lambda i: (0, i, 0)),
            pl.BlockSpec((_BLK, NWORK), lambda i: (i, 0)),
            pl.BlockSpec((1, H), lambda i: (0, 0)),
            pl.BlockSpec((wo, H), lambda i: (0, 0)),
            pl.BlockSpec((wo, H), lambda i: (0, 0)),
        ],
        out_specs=[
            pl.BlockSpec((_BLK, wo), lambda i: (i, 0)),
            pl.BlockSpec((_BLK, wo), lambda i: (i, 0)),
        ],
        out_shape=[jax.ShapeDtypeStruct((N, wo), jnp.float32),
                   jax.ShapeDtypeStruct((N, wo), jnp.float32)],
    )(s, aggp, degp, b, wn, ws)


def _tail_body(s_ref, agg_ref, deg_ref, b_ref, o_ref):
    agg = agg_ref[0] + agg_ref[1]
    deg = jnp.sum(deg_ref[...], axis=1, keepdims=True)
    inv = 1.0 / jnp.maximum(deg, 1.0)
    o_ref[...] = s_ref[...] + agg * inv + b_ref[...]


def _tail(s, aggp, degp, b):
    wo = s.shape[1]
    return pl.pallas_call(
        _tail_body,
        grid=(_GRID,),
        in_specs=[
            pl.BlockSpec((_BLK, wo), lambda i: (i, 0)),
            pl.BlockSpec((NC, _BLK, wo), lambda i: (0, i, 0)),
            pl.BlockSpec((_BLK, NWORK), lambda i: (i, 0)),
            pl.BlockSpec((1, wo), lambda i: (0, 0)),
        ],
        out_specs=pl.BlockSpec((_BLK, wo), lambda i: (i, 0)),
        out_shape=jax.ShapeDtypeStruct((N, wo), jnp.float32),
    )(s, aggp, degp, b)


def kernel(features, edge_index, Wself0, Wneigh0, b0, Wself1, Wneigh1, b1,
           Wself2, Wneigh2, b2):
    ei = edge_index.astype(jnp.int32)
    srcA = ei[0].reshape(NWORK * NSEG, SEG, CHB)
    dstA = ei[1].reshape(NWORK * NSEG, SEG, CHB)
    dst2d = ei[1].reshape(NWORK, NCHUNK, CH)

    wn2p = jnp.zeros((H, H), jnp.float32).at[:C].set(Wneigh2)
    ws2p = jnp.zeros((H, H), jnp.float32).at[:C].set(Wself2)
    b2p = jnp.zeros((1, H), jnp.float32).at[0, :C].set(b2)
    b0r = b0.reshape(1, H)
    b1r = b1.reshape(1, H)

    degp = _sc_deg(dst2d).T
    y0, s0 = _mm0(features, Wneigh0, Wself0)
    agg0 = _sc_agg(y0, srcA, dstA)
    y1, s1 = _mid(s0, agg0, degp, b0r, Wneigh1, Wself1)
    agg1 = _sc_agg(y1, srcA, dstA)
    y2, s2 = _mid(s1, agg1, degp, b1r, wn2p, ws2p)
    agg2 = _sc_agg(y2, srcA, dstA)
    out = _tail(s2, agg2, degp, b2p)
    return out[:, :C]


# speedup vs baseline: 1.1483x; 1.1483x over previous
import jax
import jax.numpy as jnp
from jax import lax
from jax.experimental import pallas as pl
from jax.experimental.pallas import tpu as pltpu
from jax.experimental.pallas import tpu_sc as plsc

N = 10000
E = 320000
D = 128
H = 128
C = 47

NC = 2
NS = 16
NWORK = NC * NS
EPW = E // NWORK
CH = 80
NCHUNK = EPW // CH
PUB = 1000
NPUB = N // PUB
ZR = 40
NZ = PUB // ZR


CHB = 100
NCB = EPW // CHB
NSEG = 10
SEG = NCB // NSEG
NBUF = 3


def _sc_agg_body(y_hbm, src_hbm, dst_hbm, out_hbm, srcv, dstv, rows, semg,
                 sems, semi, agg_sh):
    cid = lax.axis_index("c")
    sid = lax.axis_index("s")
    wid = cid * NS + sid

    def _load_idx(q, h):
        pltpu.async_copy(src_hbm.at[wid * NSEG + h], srcv.at[q], semi.at[q])
        pltpu.async_copy(dst_hbm.at[wid * NSEG + h], dstv.at[q], semi.at[q])

    def _wait_idx(q):
        pltpu.make_async_copy(src_hbm.at[0], srcv.at[q], semi.at[q]).wait()
        pltpu.make_async_copy(dst_hbm.at[0], dstv.at[q], semi.at[q]).wait()

    _load_idx(0, 0)

    def _zero_row(i, _):
        for j in range(H // 16):
            rows[0, i, pl.ds(j * 16, 16)] = jnp.zeros((16,), jnp.float32)
        return 0
    lax.fori_loop(0, ZR, _zero_row, 0)
    r0 = sid * PUB

    @pl.when(sid < NPUB)
    def _():
        for k in range(NZ):
            pltpu.sync_copy(rows.at[0, pl.ds(0, ZR), :],
                            agg_sh.at[pl.ds(r0 + k * ZR, ZR), :])
    plsc.subcore_barrier()
    _wait_idx(0)

    def _startg(b, q, j):
        pltpu.async_copy(y_hbm.at[srcv.at[q, j]], rows.at[b], semg.at[b])

    def _waitg(b):
        pltpu.make_async_copy(y_hbm.at[srcv.at[0, 0]], rows.at[b],
                              semg.at[b]).wait()

    def _scat(b, q, j):
        pltpu.async_copy(rows.at[b], agg_sh.at[dstv.at[q, j]], sems.at[b],
                         add=True)

    def _waits(b):
        pltpu.make_async_copy(rows.at[b], agg_sh.at[dstv.at[0, 0]],
                              sems.at[b]).wait()

    def _body(c, _):
        @pl.when(c < NCB)
        def _():
            b = lax.rem(c, NBUF)
            h = lax.div(c, SEG)
            q = lax.rem(h, 2)
            j = lax.rem(c, SEG)

            @pl.when(c >= NBUF)
            def _():
                _waits(b)

            @pl.when(jnp.logical_and(j == 0, c > 0))
            def _():
                _wait_idx(q)

            @pl.when(jnp.logical_and(j == 3, h < NSEG - 1))
            def _():
                _load_idx(lax.rem(h + 1, 2), h + 1)
            _startg(b, q, j)

        @pl.when(c >= 2)
        def _():
            c2 = c - 2
            b2 = lax.rem(c2, NBUF)
            q2 = lax.rem(lax.div(c2, SEG), 2)
            j2 = lax.rem(c2, SEG)
            _waitg(b2)
            _scat(b2, q2, j2)
        return 0
    lax.fori_loop(0, NCB + 2, _body, 0)
    for b in range(NBUF):
        _waits(b)
    plsc.subcore_barrier()

    @pl.when(sid < NPUB)
    def _():
        sl = pl.ds(r0, PUB)
        pltpu.sync_copy(agg_sh.at[sl, :], out_hbm.at[cid, sl, :])


_sc_agg = pl.kernel(
    _sc_agg_body,
    out_type=jax.ShapeDtypeStruct((NC, N, H), jnp.float32),
    mesh=plsc.VectorSubcoreMesh(core_axis_name="c", subcore_axis_name="s"),
    scratch_types=[
        pltpu.VMEM((2, SEG, CHB), jnp.int32),
        pltpu.VMEM((2, SEG, CHB), jnp.int32),
        pltpu.VMEM((NBUF, CHB, H), jnp.float32),
        pltpu.SemaphoreType.DMA((NBUF,)),
        pltpu.SemaphoreType.DMA((NBUF,)),
        pltpu.SemaphoreType.DMA((2,)),
        pltpu.VMEM_SHARED((N, H), jnp.float32),
    ],
)


def _sc_deg_body(dst_hbm, deg_hbm, dstv, hist):
    cid = lax.axis_index("c")
    sid = lax.axis_index("s")
    wid = cid * NS + sid

    def _zero(i, _):
        hist[pl.ds(i * 16, 16)] = jnp.zeros((16,), jnp.float32)
        return 0
    lax.fori_loop(0, N // 16, _zero, 0)

    pltpu.sync_copy(dst_hbm.at[wid], dstv)
    ones = jnp.ones((16,), jnp.float32)

    def _edges(i, _):
        def _vec(k, _):
            idx = dstv[i, pl.ds(k * 16, 16)]
            plsc.addupdate_scatter(hist, [idx], ones)
            return 0
        lax.fori_loop(0, CH // 16, _vec, 0)
        return 0
    lax.fori_loop(0, NCHUNK, _edges, 0)

    pltpu.sync_copy(hist, deg_hbm.at[wid])


_sc_deg = pl.kernel(
    _sc_deg_body,
    out_type=jax.ShapeDtypeStruct((NWORK, N), jnp.float32),
    mesh=plsc.VectorSubcoreMesh(core_axis_name="c", subcore_axis_name="s"),
    scratch_types=[
        pltpu.VMEM((NCHUNK, CH), jnp.int32),
        pltpu.VMEM((N,), jnp.float32),
    ],
    compiler_params=pltpu.CompilerParams(needs_layout_passes=False),
)


_BLK = 1000
_GRID = N // _BLK


def _mm0_body(x_ref, wn_ref, ws_ref, y_ref, s_ref):
    x = x_ref[...]
    y_ref[...] = jnp.dot(x, wn_ref[...].T, preferred_element_type=jnp.float32)
    s_ref[...] = jnp.dot(x, ws_ref[...].T, preferred_element_type=jnp.float32)


def _mm0(x, wn, ws):
    return pl.pallas_call(
        _mm0_body,
        grid=(_GRID,),
        in_specs=[
            pl.BlockSpec((_BLK, D), lambda i: (i, 0)),
            pl.BlockSpec((H, D), lambda i: (0, 0)),
            pl.BlockSpec((H, D), lambda i: (0, 0)),
        ],
        out_specs=[
            pl.BlockSpec((_BLK, H), lambda i: (i, 0)),
            pl.BlockSpec((_BLK, H), lambda i: (i, 0)),
        ],
        out_shape=[jax.ShapeDtypeStruct((N, H), jnp.float32),
                   jax.ShapeDtypeStruct((N, H), jnp.float32)],
    )(x, wn, ws)


def _mid_body(s_ref, agg_ref, deg_ref, b_ref, wn_ref, ws_ref, y_ref, sn_ref):
    agg = agg_ref[0] + agg_ref[1]
    deg = jnp.sum(deg_ref[...], axis=1, keepdims=True)
    inv = 1.0 / jnp.maximum(deg, 1.0)
    h = jnp.maximum(s_ref[...] + agg * inv + b_ref[...], 0.0)
    y_ref[...] = jnp.dot(h, wn_ref[...].T, preferred_element_type=jnp.float32)
    sn_ref[...] = jnp.dot(h, ws_ref[...].T, preferred_element_type=jnp.float32)


def _mid(s, aggp, degp, b, wn, ws):
    wo = wn.shape[0]
    return pl.pallas_call(
        _mid_body,
        grid=(_GRID,),
        in_specs=[
            pl.BlockSpec((_BLK, H), lambda i: (i, 0)),
            pl.BlockSpec((NC, _BLK, H), lambda i: (0, i, 0)),
            pl.BlockSpec((_BLK, NWORK), lambda i: (i, 0)),
            pl.BlockSpec((1, H), lambda i: (0, 0)),
            pl.BlockSpec((wo, H), lambda i: (0, 0)),
            pl.BlockSpec((wo, H), lambda i: (0, 0)),
        ],
        out_specs=[
            pl.BlockSpec((_BLK, wo), lambda i: (i, 0)),
            pl.BlockSpec((_BLK, wo), lambda i: (i, 0)),
        ],
        out_shape=[jax.ShapeDtypeStruct((N, wo), jnp.float32),
                   jax.ShapeDtypeStruct((N, wo), jnp.float32)],
    )(s, aggp, degp, b, wn, ws)


def _tail_body(s_ref, agg_ref, deg_ref, b_ref, o_ref):
    agg = agg_ref[0] + agg_ref[1]
    deg = jnp.sum(deg_ref[...], axis=1, keepdims=True)
    inv = 1.0 / jnp.maximum(deg, 1.0)
    o_ref[...] = s_ref[...] + agg * inv + b_ref[...]


def _tail(s, aggp, degp, b):
    wo = s.shape[1]
    return pl.pallas_call(
        _tail_body,
        grid=(_GRID,),
        in_specs=[
            pl.BlockSpec((_BLK, wo), lambda i: (i, 0)),
            pl.BlockSpec((NC, _BLK, wo), lambda i: (0, i, 0)),
            pl.BlockSpec((_BLK, NWORK), lambda i: (i, 0)),
            pl.BlockSpec((1, wo), lambda i: (0, 0)),
        ],
        out_specs=pl.BlockSpec((_BLK, wo), lambda i: (i, 0)),
        out_shape=jax.ShapeDtypeStruct((N, wo), jnp.float32),
    )(s, aggp, degp, b)


def kernel(features, edge_index, Wself0, Wneigh0, b0, Wself1, Wneigh1, b1,
           Wself2, Wneigh2, b2):
    ei = edge_index.astype(jnp.int32)
    srcA = ei[0].reshape(NWORK * NSEG, SEG, CHB)
    dstA = ei[1].reshape(NWORK * NSEG, SEG, CHB)
    dst2d = ei[1].reshape(NWORK, NCHUNK, CH)

    wn2p = jnp.zeros((H, H), jnp.float32).at[:C].set(Wneigh2)
    ws2p = jnp.zeros((H, H), jnp.float32).at[:C].set(Wself2)
    b2p = jnp.zeros((1, H), jnp.float32).at[0, :C].set(b2)
    b0r = b0.reshape(1, H)
    b1r = b1.reshape(1, H)

    degp = _sc_deg(dst2d).T
    y0, s0 = _mm0(features, Wneigh0, Wself0)
    agg0 = _sc_agg(y0, srcA, dstA)
    y1, s1 = _mid(s0, agg0, degp, b0r, Wneigh1, Wself1)
    agg1 = _sc_agg(y1, srcA, dstA)
    y2, s2 = _mid(s1, agg1, degp, b1r, wn2p, ws2p)
    agg2 = _sc_agg(y2, srcA, dstA)
    out = _tail(s2, agg2, degp, b2p)
    return out[:, :C]
